# baseline (speedup 1.0000x reference)
import functools

import jax
import jax.numpy as jnp
from jax import lax
from jax.experimental import pallas as pl
from jax.experimental.pallas import tpu as pltpu
from jax.experimental.pallas import tpu_sc as plsc

_N = 10000
_E = 320000
_H = 256
_HH = 128
_NSUB = 16
_NP = 10240
_RPT = _NP // _NSUB
_K = 125
_NCH = (_E // _NSUB) // _K
_BLK = 16
_DNCH = (_E // (2 * _NSUB)) // _K
_NTRAIN = 5000
_TPAD = 5120
_GPT = _TPAD // 32

_MESH = plsc.VectorSubcoreMesh(core_axis_name="c", subcore_axis_name="s")


@functools.partial(
    pl.kernel,
    out_type=jax.ShapeDtypeStruct((2, _NP, _HH), jnp.float32),
    mesh=_MESH,
    scratch_types=[
        pltpu.VMEM((_DNCH, _K), jnp.int32),
        pltpu.VMEM((_K, _HH), jnp.float32),
        pltpu.VMEM_SHARED((_NP, _HH), jnp.float32),
    ],
)
def _sc_degree(dst_hbm, ones_hbm, zeros_hbm, part_hbm, dstv, onesv, acc):
    c = lax.axis_index("c")
    s = lax.axis_index("s")
    rs = pl.ds(s * _RPT, _RPT)
    pltpu.sync_copy(zeros_hbm, acc.at[rs])
    pltpu.sync_copy(ones_hbm, onesv)
    pltpu.sync_copy(dst_hbm.at[c].at[s], dstv)
    plsc.subcore_barrier()

    def body(j, carry):
        pltpu.sync_copy(onesv, acc.at[dstv.at[j]], add=True)
        return carry

    lax.fori_loop(0, _DNCH, body, 0)
    plsc.subcore_barrier()
    pltpu.sync_copy(acc.at[rs], part_hbm.at[c].at[rs])


@functools.partial(
    pl.kernel,
    out_type=(
        jax.ShapeDtypeStruct((_NP, _HH), jnp.float32),
        jax.ShapeDtypeStruct((_NP, _HH), jnp.float32),
    ),
    mesh=_MESH,
    scratch_types=[
        pltpu.VMEM((_BLK, _K), jnp.int32),
        pltpu.VMEM((_BLK, _K), jnp.int32),
        pltpu.VMEM((_K, _HH), jnp.float32),
        pltpu.VMEM_SHARED((_NP, _HH), jnp.float32),
        pltpu.SemaphoreType.DMA,
    ],
)
def _sc_scatter(uL, uR, src_hbm, dst_hbm, aggL, aggR, srcv, dstv, rows,
                acc, sem):
    c = lax.axis_index("c")
    s = lax.axis_index("s")
    rs = pl.ds(s * _RPT, _RPT)

    def run(u_hbm, agg_hbm):
        pltpu.sync_copy(u_hbm.at[rs], acc.at[rs])
        plsc.subcore_barrier()

        def oblk(bk, carry):
            pltpu.sync_copy(src_hbm.at[s].at[pl.ds(bk * _BLK, _BLK)], srcv)
            pltpu.sync_copy(dst_hbm.at[s].at[pl.ds(bk * _BLK, _BLK)], dstv)
            def body(j, carry2):
                pltpu.async_copy(u_hbm.at[srcv.at[j]], rows, sem).wait()
                pltpu.sync_copy(rows, acc.at[dstv.at[j]], add=True)
                return carry2

            lax.fori_loop(0, _BLK, body, 0)
            return carry

        lax.fori_loop(0, _NCH // _BLK, oblk, 0)
        plsc.subcore_barrier()
        pltpu.sync_copy(acc.at[rs], agg_hbm.at[rs])

    @pl.when(c == 0)
    def _():
        run(uL, aggL)

    @pl.when(c == 1)
    def _():
        run(uR, aggR)


@functools.partial(
    pl.kernel,
    out_type=jax.ShapeDtypeStruct((_TPAD, _H), jnp.float32),
    mesh=_MESH,
    scratch_types=[
        pltpu.VMEM((2, _GPT // 2), jnp.int32),
        pltpu.VMEM((_GPT // 2, _H), jnp.float32),
        pltpu.SemaphoreType.DMA,
    ],
)
def _sc_gather(o_hbm, tid_hbm, out_hbm, tv, rows, sem):
    c = lax.axis_index("c")
    s = lax.axis_index("s")
    w = s * 2 + c
    pltpu.sync_copy(tid_hbm.at[w], tv)

    def body(k, carry):
        pltpu.async_copy(o_hbm.at[tv.at[k]], rows, sem).wait()
        pltpu.sync_copy(rows, out_hbm.at[pl.ds(w * _GPT + k * (_GPT // 2),
                                               _GPT // 2)])
        return carry

    lax.fori_loop(0, 2, body, 0)


def _tc0_body(part, x, w1, dinv, uL, uR):
    deg = part[0, :, 0:1] + part[1, :, 0:1] + 1.0
    di = lax.rsqrt(deg)
    dinv[...] = di
    h = jnp.dot(x[...], w1[...], preferred_element_type=jnp.float32)
    u = di * h
    uL[...] = u[:, :_HH]
    uR[...] = u[:, _HH:]


_tc0 = pl.pallas_call(
    _tc0_body,
    out_shape=(
        jax.ShapeDtypeStruct((_NP, 1), jnp.float32),
        jax.ShapeDtypeStruct((_NP, _HH), jnp.float32),
        jax.ShapeDtypeStruct((_NP, _HH), jnp.float32),
    ),
)


def _bn_relu(aggL, aggR, dinv, b, g, bt):
    z = jnp.concatenate([aggL[...], aggR[...]], axis=1) * dinv + b[...]
    zt = z[:_N]
    m = jnp.mean(zt, axis=0, keepdims=True)
    v = jnp.mean((zt - m) * (zt - m), axis=0, keepdims=True)
    return jnp.maximum((z - m) * lax.rsqrt(v + 1e-5) * g[...] + bt[...], 0.0)


def _tcmid_body(aggL, aggR, dinv_ref, b, g, bt, w, uL, uR):
    di = dinv_ref[...]
    y = _bn_relu(aggL, aggR, di, b, g, bt)
    u = di * jnp.dot(y, w[...], preferred_element_type=jnp.float32)
    uL[...] = u[:, :_HH]
    uR[...] = u[:, _HH:]


_tcmid = pl.pallas_call(
    _tcmid_body,
    out_shape=(
        jax.ShapeDtypeStruct((_NP, _HH), jnp.float32),
        jax.ShapeDtypeStruct((_NP, _HH), jnp.float32),
    ),
)


def _tcbn3_body(aggL, aggR, dinv_ref, b, g, bt, y_ref):
    y_ref[...] = _bn_relu(aggL, aggR, dinv_ref[...], b, g, bt)


_tcbn3 = pl.pallas_call(
    _tcbn3_body,
    out_shape=jax.ShapeDtypeStruct((_NP, _H), jnp.float32),
)


def _tchead_body(yt, lw, lb, fw, fb, o_ref):
    h4 = jnp.maximum(jnp.dot(yt[...], lw[...], preferred_element_type=jnp.float32) + lb[...], 0.0)
    o_ref[...] = jnp.dot(h4, fw[...], preferred_element_type=jnp.float32) + fb[...]


_tchead = pl.pallas_call(
    _tchead_body,
    out_shape=jax.ShapeDtypeStruct((_TPAD, 1), jnp.float32),
)


def kernel(x, edge_index, train_node_id, W1, b1, W2, b2, W3, b3,
           g1, bt1, g2, bt2, g3, bt3, lW, lb, fW, fb):
    src3 = edge_index[0].reshape(_NSUB, _NCH, _K)
    dst3 = edge_index[1].reshape(_NSUB, _NCH, _K)
    dstd = edge_index[1].reshape(2, _NSUB, _DNCH, _K)
    ones = jnp.ones((_K, _HH), jnp.float32)
    zer = jnp.zeros((_RPT, _HH), jnp.float32)

    part = _sc_degree(dstd, ones, zer)
    xp = jnp.pad(x, ((0, _NP - _N), (0, 0)))
    dinv, uL, uR = _tc0(part, xp, W1)

    r = lambda a: a.reshape(1, -1)
    for (b, g, bt, w) in ((b1, g1, bt1, W2), (b2, g2, bt2, W3)):
        aggL, aggR = _sc_scatter(uL, uR, src3, dst3)
        uL, uR = _tcmid(aggL, aggR, dinv, r(b), r(g), r(bt), w)

    aggL, aggR = _sc_scatter(uL, uR, src3, dst3)
    y3 = _tcbn3(aggL, aggR, dinv, r(b3), r(g3), r(bt3))

    tid = jnp.concatenate(
        [train_node_id, jnp.zeros((_TPAD - _NTRAIN,), jnp.int32)]
    ).reshape(32, 2, _GPT // 2)
    y3t = _sc_gather(y3, tid)
    out = _tchead(y3t, lW, r(lb), fW, fb.reshape(1, 1))
    return out[:_NTRAIN, 0]


# speedup vs baseline: 1.1316x; 1.1316x over previous
import functools

import jax
import jax.numpy as jnp
from jax import lax
from jax.experimental import pallas as pl
from jax.experimental.pallas import tpu as pltpu
from jax.experimental.pallas import tpu_sc as plsc

_N = 10000
_E = 320000
_H = 256
_HH = 128
_NSUB = 16
_NP = 10240
_RPT = _NP // _NSUB
_K = 125
_NCH = (_E // _NSUB) // _K
_BLK = 16
_DNCH = (_E // (2 * _NSUB)) // _K
_NTRAIN = 5000
_TPAD = 5120
_GPT = _TPAD // 32

_MESH = plsc.VectorSubcoreMesh(core_axis_name="c", subcore_axis_name="s")


@functools.partial(
    pl.kernel,
    out_type=jax.ShapeDtypeStruct((2, _NP, _HH), jnp.float32),
    mesh=_MESH,
    scratch_types=[
        pltpu.VMEM((_DNCH, _K), jnp.int32),
        pltpu.VMEM((_K, _HH), jnp.float32),
        pltpu.VMEM_SHARED((_NP, _HH), jnp.float32),
    ],
)
def _sc_degree(dst_hbm, ones_hbm, zeros_hbm, part_hbm, dstv, onesv, acc):
    c = lax.axis_index("c")
    s = lax.axis_index("s")
    rs = pl.ds(s * _RPT, _RPT)
    pltpu.sync_copy(zeros_hbm, acc.at[rs])
    pltpu.sync_copy(ones_hbm, onesv)
    pltpu.sync_copy(dst_hbm.at[c].at[s], dstv)
    plsc.subcore_barrier()

    def body(j, carry):
        pltpu.sync_copy(onesv, acc.at[dstv.at[j]], add=True)
        return carry

    lax.fori_loop(0, _DNCH, body, 0)
    plsc.subcore_barrier()
    pltpu.sync_copy(acc.at[rs], part_hbm.at[c].at[rs])


@functools.partial(
    pl.kernel,
    out_type=(
        jax.ShapeDtypeStruct((_NP, _HH), jnp.float32),
        jax.ShapeDtypeStruct((_NP, _HH), jnp.float32),
    ),
    mesh=_MESH,
    scratch_types=[
        pltpu.VMEM((_BLK, _K), jnp.int32),
        pltpu.VMEM((_BLK, _K), jnp.int32),
        pltpu.VMEM((_K, _HH), jnp.float32),
        pltpu.VMEM((_K, _HH), jnp.float32),
        pltpu.VMEM_SHARED((_NP, _HH), jnp.float32),
        pltpu.SemaphoreType.DMA,
        pltpu.SemaphoreType.DMA,
    ],
)
def _sc_scatter(uL, uR, src_hbm, dst_hbm, aggL, aggR, srcv, dstv, rows,
                rows2, acc, sem, ssem):
    c = lax.axis_index("c")
    s = lax.axis_index("s")
    rs = pl.ds(s * _RPT, _RPT)

    def run(u_hbm, agg_hbm):
        pltpu.sync_copy(u_hbm.at[rs], acc.at[rs])
        plsc.subcore_barrier()

        def oblk(bk, carry):
            pltpu.sync_copy(src_hbm.at[s].at[pl.ds(bk * _BLK, _BLK)], srcv)
            pltpu.sync_copy(dst_hbm.at[s].at[pl.ds(bk * _BLK, _BLK)], dstv)
            def body(j2, carry2):
                j = j2 * 2
                pltpu.async_copy(u_hbm.at[srcv.at[j]], rows, sem).wait()
                sd = pltpu.async_copy(rows, acc.at[dstv.at[j]], ssem,
                                      add=True)
                gd = pltpu.async_copy(u_hbm.at[srcv.at[j + 1]], rows2, sem)
                sd.wait()
                gd.wait()
                pltpu.sync_copy(rows2, acc.at[dstv.at[j + 1]], add=True)
                return carry2

            lax.fori_loop(0, _BLK // 2, body, 0)
            return carry

        lax.fori_loop(0, _NCH // _BLK, oblk, 0)
        plsc.subcore_barrier()
        pltpu.sync_copy(acc.at[rs], agg_hbm.at[rs])

    @pl.when(c == 0)
    def _():
        run(uL, aggL)

    @pl.when(c == 1)
    def _():
        run(uR, aggR)


@functools.partial(
    pl.kernel,
    out_type=jax.ShapeDtypeStruct((_TPAD, _H), jnp.float32),
    mesh=_MESH,
    scratch_types=[
        pltpu.VMEM((2, _GPT // 2), jnp.int32),
        pltpu.VMEM((_GPT // 2, _H), jnp.float32),
        pltpu.SemaphoreType.DMA,
    ],
)
def _sc_gather(o_hbm, tid_hbm, out_hbm, tv, rows, sem):
    c = lax.axis_index("c")
    s = lax.axis_index("s")
    w = s * 2 + c
    pltpu.sync_copy(tid_hbm.at[w], tv)

    def body(k, carry):
        pltpu.async_copy(o_hbm.at[tv.at[k]], rows, sem).wait()
        pltpu.sync_copy(rows, out_hbm.at[pl.ds(w * _GPT + k * (_GPT // 2),
                                               _GPT // 2)])
        return carry

    lax.fori_loop(0, 2, body, 0)


def _tc0_body(part, x, w1, dinv, uL, uR):
    deg = part[0, :, 0:1] + part[1, :, 0:1] + 1.0
    di = lax.rsqrt(deg)
    dinv[...] = di
    h = jnp.dot(x[...], w1[...], preferred_element_type=jnp.float32)
    u = di * h
    uL[...] = u[:, :_HH]
    uR[...] = u[:, _HH:]


_tc0 = pl.pallas_call(
    _tc0_body,
    out_shape=(
        jax.ShapeDtypeStruct((_NP, 1), jnp.float32),
        jax.ShapeDtypeStruct((_NP, _HH), jnp.float32),
        jax.ShapeDtypeStruct((_NP, _HH), jnp.float32),
    ),
)


def _bn_relu(aggL, aggR, dinv, b, g, bt):
    z = jnp.concatenate([aggL[...], aggR[...]], axis=1) * dinv + b[...]
    zt = z[:_N]
    m = jnp.mean(zt, axis=0, keepdims=True)
    v = jnp.mean((zt - m) * (zt - m), axis=0, keepdims=True)
    return jnp.maximum((z - m) * lax.rsqrt(v + 1e-5) * g[...] + bt[...], 0.0)


def _tcmid_body(aggL, aggR, dinv_ref, b, g, bt, w, uL, uR):
    di = dinv_ref[...]
    y = _bn_relu(aggL, aggR, di, b, g, bt)
    u = di * jnp.dot(y, w[...], preferred_element_type=jnp.float32)
    uL[...] = u[:, :_HH]
    uR[...] = u[:, _HH:]


_tcmid = pl.pallas_call(
    _tcmid_body,
    out_shape=(
        jax.ShapeDtypeStruct((_NP, _HH), jnp.float32),
        jax.ShapeDtypeStruct((_NP, _HH), jnp.float32),
    ),
)


def _tcbn3_body(aggL, aggR, dinv_ref, b, g, bt, y_ref):
    y_ref[...] = _bn_relu(aggL, aggR, dinv_ref[...], b, g, bt)


_tcbn3 = pl.pallas_call(
    _tcbn3_body,
    out_shape=jax.ShapeDtypeStruct((_NP, _H), jnp.float32),
)


def _tchead_body(yt, lw, lb, fw, fb, o_ref):
    h4 = jnp.maximum(jnp.dot(yt[...], lw[...], preferred_element_type=jnp.float32) + lb[...], 0.0)
    o_ref[...] = jnp.dot(h4, fw[...], preferred_element_type=jnp.float32) + fb[...]


_tchead = pl.pallas_call(
    _tchead_body,
    out_shape=jax.ShapeDtypeStruct((_TPAD, 1), jnp.float32),
)


def kernel(x, edge_index, train_node_id, W1, b1, W2, b2, W3, b3,
           g1, bt1, g2, bt2, g3, bt3, lW, lb, fW, fb):
    src3 = edge_index[0].reshape(_NSUB, _NCH, _K)
    dst3 = edge_index[1].reshape(_NSUB, _NCH, _K)
    dstd = edge_index[1].reshape(2, _NSUB, _DNCH, _K)
    ones = jnp.ones((_K, _HH), jnp.float32)
    zer = jnp.zeros((_RPT, _HH), jnp.float32)

    part = _sc_degree(dstd, ones, zer)
    xp = jnp.pad(x, ((0, _NP - _N), (0, 0)))
    dinv, uL, uR = _tc0(part, xp, W1)

    r = lambda a: a.reshape(1, -1)
    for (b, g, bt, w) in ((b1, g1, bt1, W2), (b2, g2, bt2, W3)):
        aggL, aggR = _sc_scatter(uL, uR, src3, dst3)
        uL, uR = _tcmid(aggL, aggR, dinv, r(b), r(g), r(bt), w)

    aggL, aggR = _sc_scatter(uL, uR, src3, dst3)
    y3 = _tcbn3(aggL, aggR, dinv, r(b3), r(g3), r(bt3))

    tid = jnp.concatenate(
        [train_node_id, jnp.zeros((_TPAD - _NTRAIN,), jnp.int32)]
    ).reshape(32, 2, _GPT // 2)
    y3t = _sc_gather(y3, tid)
    out = _tchead(y3t, lW, r(lb), fW, fb.reshape(1, 1))
    return out[:_NTRAIN, 0]
